# baseline (speedup 1.0000x reference)
import functools

import jax
import jax.numpy as jnp
from jax import lax
from jax.experimental import pallas as pl
from jax.experimental.pallas import tpu as pltpu
from jax.experimental.pallas import tpu_sc as plsc

_NC = 2
_NS = 16
_L = 16


def _rsqrt16(x):
    i = plsc.bitcast(x, jnp.int32)
    i = 0x5F3759DF - lax.shift_right_arithmetic(i, 1)
    y = plsc.bitcast(i, jnp.float32)
    for _ in range(3):
        y = y * (1.5 - 0.5 * x * y * y)
    return y


def _sc_graph_vectors(edges, NT, n_real):
    EP = edges.shape[1]
    ET = EP // _NS
    G = ET // _L
    DELTA = (ET // 128) * 128
    WS = EP - (_NS - 1) * DELTA
    SKEW_G = (ET - DELTA) // _L
    SL = NT // _NS
    SLG = SL // _L
    mesh = plsc.VectorSubcoreMesh(
        core_axis_name="c", subcore_axis_name="s",
        num_cores=_NC, num_subcores=_NS)

    @functools.partial(
        pl.kernel,
        out_type=[jax.ShapeDtypeStruct((NT,), jnp.float32),
                  jax.ShapeDtypeStruct((NT,), jnp.float32)],
        mesh=mesh,
        scratch_types=[
            pltpu.VMEM((2, WS), jnp.int32),
            pltpu.VMEM((ET,), jnp.int32),
            pltpu.SemaphoreType.DMA,
            pltpu.VMEM((NT,), jnp.float32),
            pltpu.VMEM((NT,), jnp.float32),
            pltpu.VMEM((NT,), jnp.float32),
            pltpu.VMEM((_NS, SL), jnp.float32),
            pltpu.VMEM((SL,), jnp.float32),
            pltpu.VMEM((SL,), jnp.float32),
            pltpu.VMEM_SHARED((_NS, NT), jnp.float32),
            pltpu.VMEM_SHARED((NT,), jnp.float32),
        ],
        compiler_params=pltpu.CompilerParams(needs_layout_passes=False),
    )
    def run(edges_hbm, c_hbm, d_hbm,
            ew_v, pk_v, dma_sem, dinv_v, wtab_v, acc_v, comb_v, stage_v,
            stage2_v, parts_s, tab_s):
        cid = lax.axis_index("c")
        sid = lax.axis_index("s")
        is_out = cid == 0
        go = sid * SKEW_G
        edge_dma = pltpu.async_copy(
            edges_hbm.at[:, pl.ds(sid * DELTA, WS)], ew_v, dma_sem)

        def zero_acc():
            @plsc.parallel_loop(0, NT // _L, unroll=4)
            def zero_body(i):
                acc_v[pl.ds(i * _L, _L)] = jnp.zeros((_L,), jnp.float32)

        def publish_and_gather_slices():
            pltpu.sync_copy(acc_v, parts_s.at[sid])
            plsc.subcore_barrier()
            pltpu.sync_copy(parts_s.at[:, pl.ds(sid * SL, SL)], comb_v)

        def slice_total(j):
            base = j * _L
            v = comb_v[0, pl.ds(base, _L)]
            for k in range(1, _NS):
                v = v + comb_v[k, pl.ds(base, _L)]
            return v

        zero_acc()
        edge_dma.wait()
        one = jnp.ones((_L,), jnp.float32)

        @plsc.parallel_loop(0, G, unroll=10)
        def deg_body(i):
            s = ew_v[0, pl.ds((go + i) * _L, _L)]
            d = ew_v[1, pl.ds((go + i) * _L, _L)]
            pk_v[pl.ds(i * _L, _L)] = jnp.bitwise_or(d, lax.shift_left(s, 16))
            plsc.addupdate_scatter(acc_v, [d], one)

        publish_and_gather_slices()

        @plsc.parallel_loop(0, SLG, unroll=2)
        def dinv_body(j):
            deg = 1.0 + slice_total(j)
            stage_v[pl.ds(j * _L, _L)] = _rsqrt16(deg)

        pltpu.sync_copy(stage_v, tab_s.at[pl.ds(sid * SL, SL)])
        plsc.subcore_barrier()
        pltpu.sync_copy(tab_s, dinv_v)

        zero_acc()

        @plsc.parallel_loop(0, G, unroll=10)
        def s1_body(i):
            p = pk_v[pl.ds(i * _L, _L)]
            vals = plsc.load_gather(dinv_v, [p & 0xFFFF])
            plsc.addupdate_scatter(acc_v, [lax.shift_right_logical(p, 16)],
                                   vals)

        publish_and_gather_slices()
        lane = lax.iota(jnp.int32, _L)

        @plsc.parallel_loop(0, SLG, unroll=2)
        def cw_body(j):
            base = j * _L
            s1 = slice_total(j)
            dinv = dinv_v[pl.ds(sid * SL + base, _L)]
            col = sid * SL + base + lane
            c = jnp.where(col < n_real, dinv * (dinv + s1), 0.0)
            stage_v[pl.ds(base, _L)] = c
            stage2_v[pl.ds(base, _L)] = dinv * c

        @pl.when(is_out)
        def _():
            pltpu.sync_copy(stage_v, c_hbm.at[pl.ds(sid * SL, SL)])

        pltpu.sync_copy(stage2_v, tab_s.at[pl.ds(sid * SL, SL)])
        plsc.subcore_barrier()
        pltpu.sync_copy(tab_s, wtab_v)

        zero_acc()

        @plsc.parallel_loop(0, G, unroll=10)
        def s2_body(i):
            p = pk_v[pl.ds(i * _L, _L)]
            vals = plsc.load_gather(wtab_v, [p & 0xFFFF])
            plsc.addupdate_scatter(acc_v, [lax.shift_right_logical(p, 16)],
                                   vals)

        publish_and_gather_slices()

        @plsc.parallel_loop(0, SLG, unroll=2)
        def d_body(j):
            base = j * _L
            s2 = slice_total(j)
            dinv = dinv_v[pl.ds(sid * SL + base, _L)]
            w = wtab_v[pl.ds(sid * SL + base, _L)]
            stage_v[pl.ds(base, _L)] = dinv * (w + s2)

        @pl.when(is_out)
        def _():
            pltpu.sync_copy(stage_v, d_hbm.at[pl.ds(sid * SL, SL)])

    return run(edges)


def _tc_tail(c, d, x, W1, b1, W2, b2, W_out, b_out):
    n_nodes, _ = x.shape
    T = W_out.shape[1]

    D = x.shape[1]
    nb = 1
    for cand in (10, 8, 5, 4, 2):
        if n_nodes % cand == 0 and (n_nodes // cand) % 8 == 0:
            nb = cand
            break
    blk = n_nodes // nb

    def body(c_ref, d_ref, x_ref,
             W1_ref, b1_ref, W2_ref, b2_ref, Wo_ref, bo_ref, out_ref, vacc):
        i = pl.program_id(0)

        @pl.when(i == 0)
        def _():
            vacc[...] = jnp.zeros_like(vacc)

        vacc[...] += lax.dot_general(
            d_ref[...], x_ref[...], (((0,), (0,)), ((), ())),
            preferred_element_type=jnp.float32)

        @pl.when(i == nb - 1)
        def _():
            Sc = jnp.sum(c_ref[...])
            v = vacc[...]
            u = jnp.dot(v, W1_ref[...],
                        preferred_element_type=jnp.float32) + Sc * b1_ref[...]
            pooled = jnp.dot(u, W2_ref[...],
                             preferred_element_type=jnp.float32)
            pooled = pooled * (1.0 / n_nodes) + b2_ref[...]
            pred = jnp.dot(pooled, Wo_ref[...],
                           preferred_element_type=jnp.float32) + bo_ref[...]
            m = jnp.max(pred, axis=1, keepdims=True)
            lse = jnp.log(jnp.sum(jnp.exp(pred - m), axis=1,
                                  keepdims=True)) + m
            out_ref[...] = pred - lse

    NT = c.shape[0]
    return pl.pallas_call(
        body,
        grid=(nb,),
        in_specs=[
            pl.BlockSpec((1, NT), lambda i: (0, 0)),
            pl.BlockSpec((blk, 1), lambda i: (i, 0)),
            pl.BlockSpec((blk, D), lambda i: (i, 0)),
            pl.BlockSpec(W1.shape, lambda i: (0, 0)),
            pl.BlockSpec((1, D), lambda i: (0, 0)),
            pl.BlockSpec(W2.shape, lambda i: (0, 0)),
            pl.BlockSpec((1, D), lambda i: (0, 0)),
            pl.BlockSpec(W_out.shape, lambda i: (0, 0)),
            pl.BlockSpec((1, T), lambda i: (0, 0)),
        ],
        out_specs=pl.BlockSpec((1, T), lambda i: (0, 0)),
        out_shape=jax.ShapeDtypeStruct((1, T), jnp.float32),
        scratch_shapes=[pltpu.VMEM((1, D), jnp.float32)],
    )(c.reshape(1, -1), d[:n_nodes].reshape(-1, 1), x,
      W1, b1.reshape(1, -1), W2, b2.reshape(1, -1),
      W_out, b_out.reshape(1, -1))


def kernel(features, edges, W1, b1, W2, b2, W_out, b_out):
    n = features.shape[0]
    e = edges.shape[1]

    nt = ((n + 1 + 255) // 256) * 256
    chunk = _NS * _L
    ep = ((e + chunk - 1) // chunk) * chunk
    if ep != e:
        pad = jnp.full((2, ep - e), n, dtype=edges.dtype)
        edges = jnp.concatenate([edges, pad], axis=1)

    c, d = _sc_graph_vectors(edges, nt, n)
    return _tc_tail(c, d, features, W1, b1, W2, b2, W_out, b_out)


# speedup vs baseline: 1.2549x; 1.2549x over previous
import functools

import jax
import jax.numpy as jnp
from jax import lax
from jax.experimental import pallas as pl
from jax.experimental.pallas import tpu as pltpu
from jax.experimental.pallas import tpu_sc as plsc

_NC = 2
_NS = 16
_L = 16


def _rsqrt16(x):
    i = plsc.bitcast(x, jnp.int32)
    i = 0x5F3759DF - lax.shift_right_arithmetic(i, 1)
    y = plsc.bitcast(i, jnp.float32)
    for _ in range(3):
        y = y * (1.5 - 0.5 * x * y * y)
    return y


def _sc_graph_vectors(edges, NT, n_real):
    EP = edges.shape[1]
    ET = EP // _NS
    G = ET // _L
    DELTA = (ET // 128) * 128
    WS = EP - (_NS - 1) * DELTA
    SKEW_G = (ET - DELTA) // _L
    SL = NT // _NS
    SLG = SL // _L
    mesh = plsc.VectorSubcoreMesh(
        core_axis_name="c", subcore_axis_name="s",
        num_cores=_NC, num_subcores=_NS)

    @functools.partial(
        pl.kernel,
        out_type=[jax.ShapeDtypeStruct((NT,), jnp.float32),
                  jax.ShapeDtypeStruct((NT,), jnp.float32)],
        mesh=mesh,
        scratch_types=[
            pltpu.VMEM((2, WS), jnp.int32),
            pltpu.VMEM((ET,), jnp.int32),
            pltpu.SemaphoreType.DMA,
            pltpu.VMEM((NT,), jnp.float32),
            pltpu.VMEM((NT,), jnp.float32),
            pltpu.VMEM((NT,), jnp.float32),
            pltpu.VMEM((_NS, SL), jnp.float32),
            pltpu.VMEM((SL,), jnp.float32),
            pltpu.VMEM((SL,), jnp.float32),
            pltpu.VMEM_SHARED((_NS, NT), jnp.float32),
            pltpu.VMEM_SHARED((NT,), jnp.float32),
        ],
        compiler_params=pltpu.CompilerParams(needs_layout_passes=False),
    )
    def run(edges_hbm, c_hbm, d_hbm,
            ew_v, pk_v, dma_sem, dinv_v, wtab_v, acc_v, comb_v, stage_v,
            stage2_v, parts_s, tab_s):
        cid = lax.axis_index("c")
        sid = lax.axis_index("s")
        is_out = cid == 0
        go = sid * SKEW_G
        edge_dma = pltpu.async_copy(
            edges_hbm.at[:, pl.ds(sid * DELTA, WS)], ew_v, dma_sem)

        def zero_acc():
            @plsc.parallel_loop(0, NT // _L, unroll=4)
            def zero_body(i):
                acc_v[pl.ds(i * _L, _L)] = jnp.zeros((_L,), jnp.float32)

        def publish_and_gather_slices():
            pltpu.sync_copy(acc_v, parts_s.at[sid])
            plsc.subcore_barrier()
            pltpu.sync_copy(parts_s.at[:, pl.ds(sid * SL, SL)], comb_v)

        def slice_total(j):
            base = j * _L
            v = comb_v[0, pl.ds(base, _L)]
            for k in range(1, _NS):
                v = v + comb_v[k, pl.ds(base, _L)]
            return v

        zero_acc()
        edge_dma.wait()
        one = jnp.ones((_L,), jnp.float32)

        @plsc.parallel_loop(0, G, unroll=4)
        def deg_body(i):
            s = ew_v[0, pl.ds((go + i) * _L, _L)]
            d = ew_v[1, pl.ds((go + i) * _L, _L)]
            pk_v[pl.ds(i * _L, _L)] = jnp.bitwise_or(d, lax.shift_left(s, 16))
            plsc.addupdate_scatter(acc_v, [d], one)

        publish_and_gather_slices()

        @plsc.parallel_loop(0, SLG, unroll=2)
        def dinv_body(j):
            deg = 1.0 + slice_total(j)
            stage_v[pl.ds(j * _L, _L)] = _rsqrt16(deg)

        pltpu.sync_copy(stage_v, tab_s.at[pl.ds(sid * SL, SL)])
        plsc.subcore_barrier()
        pltpu.sync_copy(tab_s, dinv_v)

        zero_acc()

        @plsc.parallel_loop(0, G, unroll=4)
        def s1_body(i):
            p = pk_v[pl.ds(i * _L, _L)]
            vals = plsc.load_gather(dinv_v, [p & 0xFFFF])
            plsc.addupdate_scatter(acc_v, [lax.shift_right_logical(p, 16)],
                                   vals)

        publish_and_gather_slices()
        lane = lax.iota(jnp.int32, _L)

        @plsc.parallel_loop(0, SLG, unroll=2)
        def cw_body(j):
            base = j * _L
            s1 = slice_total(j)
            dinv = dinv_v[pl.ds(sid * SL + base, _L)]
            col = sid * SL + base + lane
            c = jnp.where(col < n_real, dinv * (dinv + s1), 0.0)
            stage_v[pl.ds(base, _L)] = c
            stage2_v[pl.ds(base, _L)] = dinv * c

        @pl.when(is_out)
        def _():
            pltpu.sync_copy(stage_v, c_hbm.at[pl.ds(sid * SL, SL)])

        pltpu.sync_copy(stage2_v, tab_s.at[pl.ds(sid * SL, SL)])
        plsc.subcore_barrier()
        pltpu.sync_copy(tab_s, wtab_v)

        zero_acc()

        @plsc.parallel_loop(0, G, unroll=4)
        def s2_body(i):
            p = pk_v[pl.ds(i * _L, _L)]
            vals = plsc.load_gather(wtab_v, [p & 0xFFFF])
            plsc.addupdate_scatter(acc_v, [lax.shift_right_logical(p, 16)],
                                   vals)

        publish_and_gather_slices()

        @plsc.parallel_loop(0, SLG, unroll=2)
        def d_body(j):
            base = j * _L
            s2 = slice_total(j)
            dinv = dinv_v[pl.ds(sid * SL + base, _L)]
            w = wtab_v[pl.ds(sid * SL + base, _L)]
            stage_v[pl.ds(base, _L)] = dinv * (w + s2)

        @pl.when(is_out)
        def _():
            pltpu.sync_copy(stage_v, d_hbm.at[pl.ds(sid * SL, SL)])

    return run(edges)


def _tc_tail(c, d, x, W1, b1, W2, b2, W_out, b_out):
    n_nodes, _ = x.shape
    T = W_out.shape[1]

    def body(c_ref, d_ref, x_ref,
             W1_ref, b1_ref, W2_ref, b2_ref, Wo_ref, bo_ref, out_ref):
        Sc = jnp.sum(c_ref[...])
        d = d_ref[...][:, :n_nodes]
        v = jnp.dot(d, x_ref[...], preferred_element_type=jnp.float32)
        u = jnp.dot(v, W1_ref[...],
                    preferred_element_type=jnp.float32) + Sc * b1_ref[...]
        pooled = jnp.dot(u, W2_ref[...],
                         preferred_element_type=jnp.float32) * (1.0 / n_nodes)
        pooled = pooled + b2_ref[...]
        pred = jnp.dot(pooled, Wo_ref[...],
                       preferred_element_type=jnp.float32) + bo_ref[...]
        m = jnp.max(pred, axis=1, keepdims=True)
        lse = jnp.log(jnp.sum(jnp.exp(pred - m), axis=1, keepdims=True)) + m
        out_ref[...] = pred - lse

    return pl.pallas_call(
        body, out_shape=jax.ShapeDtypeStruct((1, T), jnp.float32),
    )(c.reshape(1, -1), d.reshape(1, -1), x,
      W1, b1.reshape(1, -1), W2, b2.reshape(1, -1),
      W_out, b_out.reshape(1, -1))


def kernel(features, edges, W1, b1, W2, b2, W_out, b_out):
    n = features.shape[0]
    e = edges.shape[1]

    nt = ((n + 1 + 255) // 256) * 256
    chunk = _NS * _L
    ep = ((e + chunk - 1) // chunk) * chunk
    if ep != e:
        pad = jnp.full((2, ep - e), n, dtype=edges.dtype)
        edges = jnp.concatenate([edges, pad], axis=1)

    c, d = _sc_graph_vectors(edges, nt, n)
    return _tc_tail(c, d, features, W1, b1, W2, b2, W_out, b_out)
